# baseline (speedup 1.0000x reference)
import functools

import jax
import jax.numpy as jnp
from jax import lax
from jax.experimental import pallas as pl
from jax.experimental.pallas import tpu as pltpu
from jax.experimental.pallas import tpu_sc as plsc

NC, NS, L = 2, 16, 16
NW = NC * NS
NBUF = 2
SEQ_PAD = 56
MAIN = 48


def _make_emb_kernel(batch, seq, hidden):
    bat_per_w = batch // NW
    lanes = hidden // L
    tail = seq - MAIN
    mesh = plsc.VectorSubcoreMesh(core_axis_name="c", subcore_axis_name="s")

    @functools.partial(
        pl.kernel,
        mesh=mesh,
        out_type=jax.ShapeDtypeStruct((batch, seq, hidden), jnp.float32),
        scratch_types=[
            pltpu.VMEM((seq * hidden,), jnp.float32),
            pltpu.VMEM((8, hidden), jnp.float32),
        ]
        + [pltpu.VMEM((SEQ_PAD,), jnp.int32)] * NBUF
        + [pltpu.VMEM((seq, hidden), jnp.float32)] * NBUF
        + [pltpu.SemaphoreType.DMA] * (1 + 3 * NBUF),
    )
    def emb(ids_hbm, tok_hbm, pos_hbm, out_hbm, pos_v, tail_v, *refs):
        ibuf = refs[:NBUF]
        rows = refs[NBUF:2 * NBUF]
        tsem = refs[2 * NBUF]
        isem = refs[2 * NBUF + 1:3 * NBUF + 1]
        gsem = refs[3 * NBUF + 1:4 * NBUF + 1]
        ssem = refs[4 * NBUF + 1:]
        wid = lax.axis_index("s") * NC + lax.axis_index("c")
        w_base = wid * bat_per_w
        pltpu.sync_copy(pos_hbm, pos_v)

        def start_idx(g, b):
            pltpu.make_async_copy(
                ids_hbm.at[pl.ds((w_base + g) * SEQ_PAD, SEQ_PAD)], ibuf[b],
                isem[b]).start()

        def wait_idx(b):
            pltpu.make_async_copy(ids_hbm.at[pl.ds(0, SEQ_PAD)], ibuf[b],
                                  isem[b]).wait()

        def start_gather(b):
            pltpu.make_async_copy(tok_hbm.at[ibuf[b].at[pl.ds(0, MAIN)]],
                                  rows[b].at[pl.ds(0, MAIN)], gsem[b]).start()

        def wait_gather(b):
            pltpu.make_async_copy(tok_hbm.at[ibuf[b].at[pl.ds(0, MAIN)]],
                                  rows[b].at[pl.ds(0, MAIN)], gsem[b]).wait()

        def start_tail(b):
            pltpu.make_async_copy(tok_hbm.at[ibuf[b].at[pl.ds(MAIN, 8)]],
                                  tail_v, tsem).start()

        def wait_tail():
            pltpu.make_async_copy(tok_hbm.at[ibuf[0].at[pl.ds(MAIN, 8)]],
                                  tail_v, tsem).wait()

        def start_store(g, b):
            pltpu.make_async_copy(rows[b], out_hbm.at[w_base + g],
                                  ssem[b]).start()

        def wait_store(b):
            pltpu.make_async_copy(rows[b], out_hbm.at[0], ssem[b]).wait()

        def copy_tail(b):
            for r in range(tail):
                for c in range(lanes):
                    rows[b][MAIN + r, pl.ds(c * L, L)] = tail_v[r, pl.ds(c * L, L)]

        def add_pos(b):
            def row_body(r, _):
                for c in range(lanes):
                    rows[b][r, pl.ds(c * L, L)] = (
                        rows[b][r, pl.ds(c * L, L)]
                        + pos_v[pl.ds(r * hidden + c * L, L)]
                    )
                return 0

            lax.fori_loop(0, seq, row_body, 0)

        def iter_body(g, b):
            ob = 1 - b
            wait_gather(b)
            wait_tail()
            copy_tail(b)

            @pl.when(g + 1 < bat_per_w)
            def _():
                wait_idx(ob)
                start_gather(ob)
                start_tail(ob)

            @pl.when(g + 2 < bat_per_w)
            def _():
                start_idx(g + 2, b)

            add_pos(b)

            @pl.when(g >= 1)
            def _():
                wait_store(ob)

            start_store(g, b)

        start_idx(0, 0)
        start_idx(1, 1)
        wait_idx(0)
        start_gather(0)
        start_tail(0)

        def outer(o, _):
            for j in range(NBUF):
                iter_body(o * NBUF + j, j)
            return 0

        lax.fori_loop(0, bat_per_w // NBUF, outer, 0)
        wait_store((bat_per_w - 1) % NBUF)

    return emb


def kernel(input_ids, token_table, pos_table):
    batch, seq = input_ids.shape
    hidden = token_table.shape[1]
    ids_pad = jnp.pad(input_ids.astype(jnp.int32),
                      ((0, 0), (0, SEQ_PAD - seq))).reshape(-1)
    pos_flat = pos_table[:seq].reshape(-1)
    emb = _make_emb_kernel(batch, seq, hidden)
    return emb(ids_pad, token_table, pos_flat)


# speedup vs baseline: 1.2160x; 1.2160x over previous
import functools

import jax
import jax.numpy as jnp
from jax import lax
from jax.experimental import pallas as pl
from jax.experimental.pallas import tpu as pltpu
from jax.experimental.pallas import tpu_sc as plsc

NC, NS, L = 2, 16, 16
NW = NC * NS
NBUF = 2
MAIN = 48
QUAD = 4


def _make_emb_kernel(batch, seq, hidden):
    bat_per_w = batch // NW
    n_quads = bat_per_w // QUAD
    lanes = hidden // L
    tail = seq - MAIN
    t_per_w = bat_per_w * tail
    mesh = plsc.VectorSubcoreMesh(core_axis_name="c", subcore_axis_name="s")

    @functools.partial(
        pl.kernel,
        mesh=mesh,
        out_type=jax.ShapeDtypeStruct((batch, seq, hidden), jnp.float32),
        scratch_types=[
            pltpu.VMEM((seq * hidden,), jnp.float32),
            pltpu.VMEM((QUAD * tail, hidden), jnp.float32),
            pltpu.VMEM((t_per_w,), jnp.int32),
        ]
        + [pltpu.VMEM((MAIN,), jnp.int32)] * NBUF
        + [pltpu.VMEM((seq, hidden), jnp.float32)] * NBUF
        + [pltpu.SemaphoreType.DMA] * (1 + 3 * NBUF),
    )
    def emb(ids_hbm, tids_hbm, tok_hbm, pos_hbm, out_hbm, pos_v, tail_v,
            tids_v, *refs):
        ibuf = refs[:NBUF]
        rows = refs[NBUF:2 * NBUF]
        tsem = refs[2 * NBUF]
        isem = refs[2 * NBUF + 1:3 * NBUF + 1]
        gsem = refs[3 * NBUF + 1:4 * NBUF + 1]
        ssem = refs[4 * NBUF + 1:]
        wid = lax.axis_index("s") * NC + lax.axis_index("c")
        w_base = wid * bat_per_w
        pltpu.sync_copy(pos_hbm, pos_v)
        pltpu.sync_copy(tids_hbm.at[pl.ds(wid * t_per_w, t_per_w)], tids_v)

        def start_idx(g, b):
            pltpu.make_async_copy(
                ids_hbm.at[pl.ds((w_base + g) * MAIN, MAIN)], ibuf[b],
                isem[b]).start()

        def wait_idx(b):
            pltpu.make_async_copy(ids_hbm.at[pl.ds(0, MAIN)], ibuf[b],
                                  isem[b]).wait()

        def start_gather(b):
            pltpu.make_async_copy(tok_hbm.at[ibuf[b]],
                                  rows[b].at[pl.ds(0, MAIN)], gsem[b]).start()

        def wait_gather(b):
            pltpu.make_async_copy(tok_hbm.at[ibuf[b]],
                                  rows[b].at[pl.ds(0, MAIN)], gsem[b]).wait()

        def start_tail(q):
            pltpu.make_async_copy(
                tok_hbm.at[tids_v.at[pl.ds(q * QUAD * tail, QUAD * tail)]],
                tail_v, tsem).start()

        def wait_tail():
            pltpu.make_async_copy(tok_hbm.at[tids_v.at[pl.ds(0, QUAD * tail)]],
                                  tail_v, tsem).wait()

        def start_store(g, b):
            pltpu.make_async_copy(rows[b], out_hbm.at[w_base + g],
                                  ssem[b]).start()

        def wait_store(b):
            pltpu.make_async_copy(rows[b], out_hbm.at[0], ssem[b]).wait()

        def copy_tail(b, j):
            for r in range(tail):
                for c in range(lanes):
                    rows[b][MAIN + r, pl.ds(c * L, L)] = tail_v[
                        j * tail + r, pl.ds(c * L, L)]

        def add_pos(b):
            def row_body(r, _):
                for c in range(lanes):
                    rows[b][r, pl.ds(c * L, L)] = (
                        rows[b][r, pl.ds(c * L, L)]
                        + pos_v[pl.ds(r * hidden + c * L, L)]
                    )
                return 0

            lax.fori_loop(0, seq, row_body, 0)

        def iter_body(q, j, g, b):
            ob = 1 - b
            wait_gather(b)
            if j == 0:
                wait_tail()
            copy_tail(b, j)

            @pl.when(g >= 1)
            def _():
                wait_store(ob)

            @pl.when(g + 1 < bat_per_w)
            def _():
                wait_idx(ob)
                start_gather(ob)

            @pl.when(g + 2 < bat_per_w)
            def _():
                start_idx(g + 2, b)

            if j == QUAD - 1:
                @pl.when(q + 1 < n_quads)
                def _():
                    start_tail(q + 1)

            add_pos(b)
            start_store(g, b)

        start_idx(0, 0)
        start_idx(1, 1)
        wait_idx(0)
        start_gather(0)
        start_tail(0)

        def outer(q, _):
            for j in range(QUAD):
                g = q * QUAD + j
                iter_body(q, j, g, j % NBUF)
            return 0

        lax.fori_loop(0, n_quads, outer, 0)
        wait_store((bat_per_w - 1) % NBUF)

    return emb


def kernel(input_ids, token_table, pos_table):
    batch, seq = input_ids.shape
    hidden = token_table.shape[1]
    ids32 = input_ids.astype(jnp.int32)
    ids_main = ids32[:, :MAIN].reshape(-1)
    ids_tail = ids32[:, MAIN:].reshape(-1)
    pos_flat = pos_table[:seq].reshape(-1)
    emb = _make_emb_kernel(batch, seq, hidden)
    return emb(ids_main, ids_tail, token_table, pos_flat)
